# baseline (speedup 1.0000x reference)
import functools

import jax
import jax.numpy as jnp
from jax import lax
from jax.experimental import pallas as pl
from jax.experimental.pallas import tpu as pltpu
from jax.experimental.pallas import tpu_sc as plsc

_CW = 8192
_CHUNK = 128
_LANES = 16


def _pair_body(a_ref, b_ref, z_ref):
    stacked = jnp.concatenate([a_ref[...], b_ref[...]], axis=0)
    z_ref[...] = jnp.transpose(stacked, (1, 0))


@functools.lru_cache(maxsize=None)
def _make_pair(V, D):
    H = ((V // 2) + _CW - 1) // _CW * _CW
    n_steps = H // _CW
    hb = H // _CW
    last_b = (V + _CW - 1) // _CW - 1
    return pl.pallas_call(
        _pair_body,
        grid=(n_steps,),
        in_specs=[
            pl.BlockSpec((D, _CW), lambda i: (0, i)),
            pl.BlockSpec((D, _CW), lambda i: (0, jnp.minimum(i + hb, last_b))),
        ],
        out_specs=pl.BlockSpec((_CW, 2 * D), lambda i: (i, 0)),
        out_shape=jax.ShapeDtypeStruct((H, 2 * D), jnp.float32),
    )


@functools.lru_cache(maxsize=None)
def _make_gather(H, B, NC, NS):
    NW = NC * NS
    b_per_w = B // NW
    n_ch = b_per_w // _CHUNK
    mesh = plsc.VectorSubcoreMesh(core_axis_name="c", subcore_axis_name="s")

    @functools.partial(
        pl.kernel,
        out_type=jax.ShapeDtypeStruct((B, 128), jnp.float32),
        mesh=mesh,
        scratch_types=[
            pltpu.VMEM((n_ch, _CHUNK), jnp.int32),
            pltpu.VMEM((n_ch, _CHUNK), jnp.int32),
            pltpu.VMEM((b_per_w,), jnp.int32),
            pltpu.VMEM((b_per_w, 128), jnp.float32),
            pltpu.SemaphoreType.DMA,
        ],
        compiler_params=pltpu.CompilerParams(
            use_tc_tiling_on_sc=True, needs_layout_passes=False
        ),
    )
    def gather_kernel(z_hbm, idx_hbm, out_hbm, idx_v, pair_v, half_v, rows_v, sem):
        wid = lax.axis_index("s") * NC + lax.axis_index("c")
        base = wid * b_per_w
        pltpu.sync_copy(idx_hbm.at[wid], idx_v)

        for g in range(n_ch):
            for l in range(_CHUNK // _LANES):
                v = idx_v[g, pl.ds(l * _LANES, _LANES)]
                m = (v >= H).astype(jnp.int32)
                pair_v[g, pl.ds(l * _LANES, _LANES)] = v - m * H
                half_v[pl.ds(g * _CHUNK + l * _LANES, _LANES)] = m

        copies = [
            pltpu.async_copy(
                z_hbm.at[pair_v.at[g]],
                rows_v.at[pl.ds(g * _CHUNK, _CHUNK)],
                sem,
            )
            for g in range(n_ch)
        ]
        for c in copies:
            c.wait()

        def sel_body(i, carry):
            splat = jnp.zeros((_LANES,), jnp.int32) + i
            m = plsc.load_gather(half_v, [splat]) != 0
            for k in range(64 // _LANES):
                lo = rows_v[i, pl.ds(k * _LANES, _LANES)]
                hi = rows_v[i, pl.ds(64 + k * _LANES, _LANES)]
                rows_v[i, pl.ds(k * _LANES, _LANES)] = jnp.where(m, hi, lo)
            return carry

        lax.fori_loop(0, b_per_w, sel_body, 0)
        pltpu.sync_copy(rows_v, out_hbm.at[pl.ds(base, b_per_w)])

    return gather_kernel


def kernel(x, dim, index, out):
    del dim, out
    V, D = x.shape
    B = index.shape[0]
    info = plsc.get_sparse_core_info()
    NC, NS = info.num_cores, info.num_subcores
    NW = NC * NS
    H = ((V // 2) + _CW - 1) // _CW * _CW
    xt = x.T
    z = _make_pair(V, D)(xt, xt)
    idx3 = index.astype(jnp.int32).reshape(NW, (B // NW) // _CHUNK, _CHUNK)
    y = _make_gather(H, B, NC, NS)(z, idx3)
    return y[:, :D]


# speedup vs baseline: 1.0333x; 1.0333x over previous
import functools

import jax
import jax.numpy as jnp
from jax import lax
from jax.experimental import pallas as pl
from jax.experimental.pallas import tpu as pltpu
from jax.experimental.pallas import tpu_sc as plsc

_CW = 16384
_CHUNK = 128
_LANES = 16


def _pair_body(a_ref, b_ref, z_ref):
    stacked = jnp.concatenate([a_ref[...], b_ref[...]], axis=0)
    z_ref[...] = jnp.transpose(stacked, (1, 0))


@functools.lru_cache(maxsize=None)
def _make_pair(V, D):
    H = ((V // 2) + _CW - 1) // _CW * _CW
    n_steps = H // _CW
    hb = H // _CW
    last_b = (V + _CW - 1) // _CW - 1
    return pl.pallas_call(
        _pair_body,
        grid=(n_steps,),
        in_specs=[
            pl.BlockSpec((D, _CW), lambda i: (0, i)),
            pl.BlockSpec((D, _CW), lambda i: (0, jnp.minimum(i + hb, last_b))),
        ],
        out_specs=pl.BlockSpec((_CW, 2 * D), lambda i: (i, 0)),
        out_shape=jax.ShapeDtypeStruct((H, 2 * D), jnp.float32),
    )


@functools.lru_cache(maxsize=None)
def _make_gather(H, B, NC, NS):
    NW = NC * NS
    b_per_w = B // NW
    n_ch = b_per_w // _CHUNK
    mesh = plsc.VectorSubcoreMesh(core_axis_name="c", subcore_axis_name="s")

    @functools.partial(
        pl.kernel,
        out_type=jax.ShapeDtypeStruct((B, 128), jnp.float32),
        mesh=mesh,
        scratch_types=[
            pltpu.VMEM((n_ch, _CHUNK), jnp.int32),
            pltpu.VMEM((n_ch, _CHUNK), jnp.int32),
            pltpu.VMEM((b_per_w,), jnp.int32),
            pltpu.VMEM((b_per_w, 128), jnp.float32),
            [pltpu.SemaphoreType.DMA] * (b_per_w // _CHUNK),
        ],
        compiler_params=pltpu.CompilerParams(
            use_tc_tiling_on_sc=True, needs_layout_passes=False
        ),
    )
    def gather_kernel(z_hbm, idx_hbm, out_hbm, idx_v, pair_v, half_v, rows_v, sems):
        wid = lax.axis_index("s") * NC + lax.axis_index("c")
        base = wid * b_per_w
        pltpu.sync_copy(idx_hbm.at[wid], idx_v)

        for g in range(n_ch):
            for l in range(_CHUNK // _LANES):
                v = idx_v[g, pl.ds(l * _LANES, _LANES)]
                m = (v >= H).astype(jnp.int32)
                pair_v[g, pl.ds(l * _LANES, _LANES)] = v - m * H
                half_v[pl.ds(g * _CHUNK + l * _LANES, _LANES)] = m

        copies = [
            pltpu.async_copy(
                z_hbm.at[pair_v.at[g]],
                rows_v.at[pl.ds(g * _CHUNK, _CHUNK)],
                sems[g],
            )
            for g in range(n_ch)
        ]

        def sel_body(i, carry):
            splat = jnp.zeros((_LANES,), jnp.int32) + i
            m = plsc.load_gather(half_v, [splat]) != 0
            for k in range(64 // _LANES):
                lo = rows_v[i, pl.ds(k * _LANES, _LANES)]
                hi = rows_v[i, pl.ds(64 + k * _LANES, _LANES)]
                rows_v[i, pl.ds(k * _LANES, _LANES)] = jnp.where(m, hi, lo)
            return carry

        for g in range(n_ch):
            copies[g].wait()
            lax.fori_loop(g * _CHUNK, (g + 1) * _CHUNK, sel_body, 0)
        pltpu.sync_copy(rows_v, out_hbm.at[pl.ds(base, b_per_w)])

    return gather_kernel


def kernel(x, dim, index, out):
    del dim, out
    V, D = x.shape
    B = index.shape[0]
    info = plsc.get_sparse_core_info()
    NC, NS = info.num_cores, info.num_subcores
    NW = NC * NS
    H = ((V // 2) + _CW - 1) // _CW * _CW
    xt = x.T
    z = _make_pair(V, D)(xt, xt)
    idx3 = index.astype(jnp.int32).reshape(NW, (B // NW) // _CHUNK, _CHUNK)
    y = _make_gather(H, B, NC, NS)(z, idx3)
    return y[:, :D]
